# Initial kernel scaffold; baseline (speedup 1.0000x reference)
import jax
import jax.numpy as jnp
from jax.experimental import pallas as pl


def kernel(x, fused_mc_w, fused_mc_b, down0_w, down0_b, up0_w, up0_b, fused_tail_w, fused_tail_b):
    raise NotImplementedError("write your pallas kernel here")


# speedup vs baseline: 10.4076x; 10.4076x over previous
import functools

import jax
import jax.numpy as jnp
from jax.experimental import pallas as pl
from jax.experimental.pallas import tpu as pltpu


_TILE_H = 16

_dot = functools.partial(
    jax.lax.dot_general,
    dimension_numbers=(((1,), (0,)), ((), ())),
    preferred_element_type=jnp.float32)


def _puca_kernel(xz_ref, w2_ref, b2_ref, ws_ref, u4a_ref, u4b_ref,
                 bmid_ref, ftw_ref, bout_ref, o_ref):
    ht = o_ref.shape[2]
    wo = o_ref.shape[3]
    r0 = pl.program_id(1) * ht

    rows_all = xz_ref[0, :, pl.ds(r0, ht + 8), :]
    taps = []
    for kh in range(3):
        for kw in range(3):
            if kh == 1 and kw == 1:
                continue
            taps.append(rows_all[:, 4 + kh:4 + kh + ht, 4 + kw:4 + kw + wo])
    xcol = jnp.concatenate(taps, axis=0)

    tc = _dot(w2_ref[...], xcol) + b2_ref[...][:, :, None]
    skip = _dot(ws_ref[...], xcol)

    t4 = tc.reshape(64, ht // 4, 4, wo)
    y = jnp.concatenate(
        [t4[:, :, 0:2, :].reshape(64, ht // 2, wo),
         t4[:, :, 2:4, :].reshape(64, ht // 2, wo)], axis=0)

    lane = jax.lax.broadcasted_iota(jnp.int32, (ht // 2, wo), 1)
    laneq = (lane % 4) // 2
    acc = None
    for cc, u_ref in ((0, u4a_ref), (1, u4b_ref)):
        z = _dot(u_ref[...], y)
        for q in range(2):
            zq = z[q * 256:(q + 1) * 256]
            sh = 2 * (q - cc)
            if sh:
                zq = jnp.roll(zq, sh, axis=2)
            zq = jnp.where(laneq == q, zq, 0.0)
            acc = zq if acc is None else acc + zq
    mid = acc + bmid_ref[...][:, None, :]

    m0 = mid[:128].reshape(128, ht // 4, 2, wo)
    m1 = mid[128:].reshape(128, ht // 4, 2, wo)
    ymid = jnp.concatenate([m0, m1], axis=2).reshape(128, ht, wo)

    out = _dot(ftw_ref[...], ymid) + skip + bout_ref[...][:, :, None]
    o_ref[0] = out


def kernel(x, fused_mc_w, fused_mc_b, down0_w, down0_b, up0_w, up0_b,
           fused_tail_w, fused_tail_b):
    B, cimg, H, W = x.shape
    p, mp = 4, 1

    xp = jnp.pad(x, ((0, 0), (0, 0), (p, p), (p, p)), mode='reflect')
    ones = jnp.ones((B, 1, H + 2 * p, W + 2 * p), x.dtype)
    xz = jnp.pad(jnp.concatenate([xp, ones], axis=1),
                 ((0, 0), (0, 0), (mp, mp), (mp, mp)))
    ca, hz, wz = cimg + 1, H + 2 * p + 2 * mp, W + 2 * p + 2 * mp
    width = fused_mc_w.shape[0]
    w2c = down0_w.shape[0]

    w2 = down0_w @ fused_mc_w
    b2 = down0_w @ fused_mc_b + down0_b
    ws = fused_tail_w @ fused_mc_w
    bout = fused_tail_w @ fused_mc_b + fused_tail_b
    u6 = up0_w.reshape(width, 2, 2, w2c, 2, 2)
    u4 = u6.transpose(2, 1, 0, 5, 4, 3).reshape(4 * width, 2, 2 * w2c)
    u4a, u4b = u4[:, 0, :], u4[:, 1, :]
    ub = up0_b.reshape(width, 2, 2).transpose(1, 0, 2).reshape(2 * width, 2)
    laneq = (jnp.arange(W) % 4) // 2
    bmid = ub[:, laneq]

    return pl.pallas_call(
        _puca_kernel,
        out_shape=jax.ShapeDtypeStruct((B, cimg, H, W), jnp.float32),
        grid=(B, H // _TILE_H),
        in_specs=[
            pl.BlockSpec((1, ca, hz, wz), lambda b, t: (b, 0, 0, 0)),
            pl.BlockSpec(w2.shape, lambda b, t: (0, 0)),
            pl.BlockSpec((w2c, 1), lambda b, t: (0, 0)),
            pl.BlockSpec(ws.shape, lambda b, t: (0, 0)),
            pl.BlockSpec(u4a.shape, lambda b, t: (0, 0)),
            pl.BlockSpec(u4b.shape, lambda b, t: (0, 0)),
            pl.BlockSpec(bmid.shape, lambda b, t: (0, 0)),
            pl.BlockSpec(fused_tail_w.shape, lambda b, t: (0, 0)),
            pl.BlockSpec((cimg, 1), lambda b, t: (0, 0)),
        ],
        out_specs=pl.BlockSpec((1, cimg, _TILE_H, W), lambda b, t: (b, 0, t, 0)),
        compiler_params=pltpu.CompilerParams(
            dimension_semantics=("parallel", "arbitrary")),
    )(xz, w2, b2.reshape(w2c, 1), ws, u4a, u4b, bmid,
      fused_tail_w, bout.reshape(cimg, 1))


# speedup vs baseline: 10.6216x; 1.0206x over previous
import functools

import jax
import jax.numpy as jnp
from jax.experimental import pallas as pl
from jax.experimental.pallas import tpu as pltpu


_TILE_H = 16

_dot = functools.partial(
    jax.lax.dot_general,
    dimension_numbers=(((1,), (0,)), ((), ())),
    preferred_element_type=jnp.float32)


def _puca_kernel(xz_ref, w2_ref, b2_ref, ws_ref, u4a_ref, u4b_ref,
                 bmid_ref, ftw_ref, bout_ref, o_ref):
    ht = o_ref.shape[2]
    wo = o_ref.shape[3]
    r0 = pl.program_id(1) * ht

    rows_all = xz_ref[0, :, pl.ds(r0, ht + 8), :]
    taps = []
    for kh in range(3):
        for kw in range(3):
            if kh == 1 and kw == 1:
                continue
            taps.append(rows_all[:, 4 + kh:4 + kh + ht, 4 + kw:4 + kw + wo])
    xcol = jnp.concatenate(taps, axis=0).astype(jnp.bfloat16)

    tc = _dot(w2_ref[...], xcol) + b2_ref[...][:, :, None]
    skip = _dot(ws_ref[...], xcol)

    t4 = tc.astype(jnp.bfloat16).reshape(64, ht // 4, 4, wo)
    y = jnp.concatenate(
        [t4[:, :, 0:2, :].reshape(64, ht // 2, wo),
         t4[:, :, 2:4, :].reshape(64, ht // 2, wo)], axis=0)

    lane = jax.lax.broadcasted_iota(jnp.int32, (ht // 2, wo), 1)
    laneq = (lane % 4) // 2
    acc = None
    for cc, u_ref in ((0, u4a_ref), (1, u4b_ref)):
        z = _dot(u_ref[...], y)
        for q in range(2):
            zq = z[q * 256:(q + 1) * 256]
            sh = 2 * (q - cc)
            if sh:
                zq = jnp.roll(zq, sh, axis=2)
            zq = jnp.where(laneq == q, zq, 0.0)
            acc = zq if acc is None else acc + zq
    mid = acc + bmid_ref[...][:, None, :]

    m0 = mid[:128].reshape(128, ht // 4, 2, wo)
    m1 = mid[128:].reshape(128, ht // 4, 2, wo)
    ymid = jnp.concatenate([m0, m1], axis=2).reshape(128, ht, wo)

    out = (_dot(ftw_ref[...], ymid.astype(jnp.bfloat16))
           + skip + bout_ref[...][:, :, None])
    o_ref[0] = out


def kernel(x, fused_mc_w, fused_mc_b, down0_w, down0_b, up0_w, up0_b,
           fused_tail_w, fused_tail_b):
    B, cimg, H, W = x.shape
    p, mp = 4, 1

    xp = jnp.pad(x, ((0, 0), (0, 0), (p, p), (p, p)), mode='reflect')
    ones = jnp.ones((B, 1, H + 2 * p, W + 2 * p), x.dtype)
    xz = jnp.pad(jnp.concatenate([xp, ones], axis=1),
                 ((0, 0), (0, 0), (mp, mp), (mp, mp)))
    ca, hz, wz = cimg + 1, H + 2 * p + 2 * mp, W + 2 * p + 2 * mp
    width = fused_mc_w.shape[0]
    w2c = down0_w.shape[0]

    w2 = down0_w @ fused_mc_w
    b2 = down0_w @ fused_mc_b + down0_b
    ws = fused_tail_w @ fused_mc_w
    bout = fused_tail_w @ fused_mc_b + fused_tail_b
    u6 = up0_w.reshape(width, 2, 2, w2c, 2, 2)
    u4 = u6.transpose(2, 1, 0, 5, 4, 3).reshape(4 * width, 2, 2 * w2c)
    u4a, u4b = u4[:, 0, :], u4[:, 1, :]
    ub = up0_b.reshape(width, 2, 2).transpose(1, 0, 2).reshape(2 * width, 2)
    laneq = (jnp.arange(W) % 4) // 2
    bmid = ub[:, laneq]

    return pl.pallas_call(
        _puca_kernel,
        out_shape=jax.ShapeDtypeStruct((B, cimg, H, W), jnp.float32),
        grid=(B, H // _TILE_H),
        in_specs=[
            pl.BlockSpec((1, ca, hz, wz), lambda b, t: (b, 0, 0, 0)),
            pl.BlockSpec(w2.shape, lambda b, t: (0, 0)),
            pl.BlockSpec((w2c, 1), lambda b, t: (0, 0)),
            pl.BlockSpec(ws.shape, lambda b, t: (0, 0)),
            pl.BlockSpec(u4a.shape, lambda b, t: (0, 0)),
            pl.BlockSpec(u4b.shape, lambda b, t: (0, 0)),
            pl.BlockSpec(bmid.shape, lambda b, t: (0, 0)),
            pl.BlockSpec(fused_tail_w.shape, lambda b, t: (0, 0)),
            pl.BlockSpec((cimg, 1), lambda b, t: (0, 0)),
        ],
        out_specs=pl.BlockSpec((1, cimg, _TILE_H, W), lambda b, t: (b, 0, t, 0)),
        compiler_params=pltpu.CompilerParams(
            dimension_semantics=("parallel", "arbitrary")),
    )(xz, w2.astype(jnp.bfloat16), b2.reshape(w2c, 1),
      ws.astype(jnp.bfloat16), u4a.astype(jnp.bfloat16),
      u4b.astype(jnp.bfloat16), bmid,
      fused_tail_w.astype(jnp.bfloat16), bout.reshape(cimg, 1))


# speedup vs baseline: 19.1149x; 1.7996x over previous
import functools

import jax
import jax.numpy as jnp
from jax.experimental import pallas as pl
from jax.experimental.pallas import tpu as pltpu


_TILE_H = 16

_dot = functools.partial(
    jax.lax.dot_general,
    dimension_numbers=(((1,), (0,)), ((), ())),
    preferred_element_type=jnp.float32)


def _puca_kernel(xz_ref, w2_ref, b2_ref, ws_ref, u4a_ref, u4b_ref,
                 bmid_ref, ftw_ref, bout_ref, o_ref):
    ht = o_ref.shape[2]
    wo = o_ref.shape[3]
    n = ht * wo
    r0 = pl.program_id(1) * ht

    rows_all = xz_ref[0, :, pl.ds(r0, ht + 8), :]
    taps = []
    for kh in range(3):
        for kw in range(3):
            if kh == 1 and kw == 1:
                continue
            taps.append(rows_all[:, 4 + kh:4 + kh + ht, 4 + kw:4 + kw + wo])
    xcol = jnp.concatenate(taps, axis=0).astype(jnp.bfloat16)
    xcol = xcol.reshape(8 * rows_all.shape[0], n)

    tcf = _dot(w2_ref[...], xcol) + b2_ref[...]
    skip = _dot(ws_ref[...], xcol)
    tcb = tcf.astype(jnp.bfloat16)

    blk = 2 * wo
    y = jnp.concatenate(
        [jnp.concatenate([tcb[:, (2 * i + a) * blk:(2 * i + a + 1) * blk]
                          for i in range(ht // 4)], axis=1)
         for a in range(2)], axis=0)

    lane = jax.lax.broadcasted_iota(jnp.int32, (256, n // 2), 1)
    laneq = (lane % 4) // 2
    acc = None
    for cc, u_ref in ((0, u4a_ref), (1, u4b_ref)):
        z = _dot(u_ref[...], y)
        for q in range(2):
            zq = z[q * 256:(q + 1) * 256]
            sh = 2 * (q - cc)
            if sh:
                zq = jnp.roll(zq, sh, axis=1)
            zq = jnp.where(laneq == q, zq, 0.0)
            acc = zq if acc is None else acc + zq
    mid = (acc + bmid_ref[...]).astype(jnp.bfloat16)

    pieces = []
    for i in range(ht // 4):
        pieces.append(mid[0:128, i * blk:(i + 1) * blk])
        pieces.append(mid[128:256, i * blk:(i + 1) * blk])
    ymid = jnp.concatenate(pieces, axis=1)

    out = _dot(ftw_ref[...], ymid) + skip + bout_ref[...]
    o_ref[0] = out.reshape(out.shape[0], ht, wo)


def kernel(x, fused_mc_w, fused_mc_b, down0_w, down0_b, up0_w, up0_b,
           fused_tail_w, fused_tail_b):
    B, cimg, H, W = x.shape
    p, mp = 4, 1

    xp = jnp.pad(x, ((0, 0), (0, 0), (p, p), (p, p)), mode='reflect')
    ones = jnp.ones((B, 1, H + 2 * p, W + 2 * p), x.dtype)
    xz = jnp.pad(jnp.concatenate([xp, ones], axis=1),
                 ((0, 0), (0, 0), (mp, mp), (mp, mp)))
    ca, hz, wz = cimg + 1, H + 2 * p + 2 * mp, W + 2 * p + 2 * mp
    width = fused_mc_w.shape[0]
    w2c = down0_w.shape[0]

    w2 = down0_w @ fused_mc_w
    b2 = down0_w @ fused_mc_b + down0_b
    ws = fused_tail_w @ fused_mc_w
    bout = fused_tail_w @ fused_mc_b + fused_tail_b
    u6 = up0_w.reshape(width, 2, 2, w2c, 2, 2)
    u4 = u6.transpose(2, 1, 0, 5, 4, 3).reshape(4 * width, 2, 2 * w2c)
    u4a, u4b = u4[:, 0, :], u4[:, 1, :]
    ub = up0_b.reshape(width, 2, 2).transpose(1, 0, 2).reshape(2 * width, 2)
    laneq = (jnp.arange(_TILE_H // 2 * W) % 4) // 2
    bmid = ub[:, laneq]

    return pl.pallas_call(
        _puca_kernel,
        out_shape=jax.ShapeDtypeStruct((B, cimg, H, W), jnp.float32),
        grid=(B, H // _TILE_H),
        in_specs=[
            pl.BlockSpec((1, ca, hz, wz), lambda b, t: (b, 0, 0, 0)),
            pl.BlockSpec(w2.shape, lambda b, t: (0, 0)),
            pl.BlockSpec((w2c, 1), lambda b, t: (0, 0)),
            pl.BlockSpec(ws.shape, lambda b, t: (0, 0)),
            pl.BlockSpec(u4a.shape, lambda b, t: (0, 0)),
            pl.BlockSpec(u4b.shape, lambda b, t: (0, 0)),
            pl.BlockSpec(bmid.shape, lambda b, t: (0, 0)),
            pl.BlockSpec(fused_tail_w.shape, lambda b, t: (0, 0)),
            pl.BlockSpec((cimg, 1), lambda b, t: (0, 0)),
        ],
        out_specs=pl.BlockSpec((1, cimg, _TILE_H, W), lambda b, t: (b, 0, t, 0)),
        compiler_params=pltpu.CompilerParams(
            dimension_semantics=("parallel", "arbitrary")),
    )(xz, w2.astype(jnp.bfloat16), b2.reshape(w2c, 1),
      ws.astype(jnp.bfloat16), u4a.astype(jnp.bfloat16),
      u4b.astype(jnp.bfloat16), bmid,
      fused_tail_w.astype(jnp.bfloat16), bout.reshape(cimg, 1))


# speedup vs baseline: 19.3791x; 1.0138x over previous
import functools

import jax
import jax.numpy as jnp
from jax.experimental import pallas as pl
from jax.experimental.pallas import tpu as pltpu


_TILE_H = 16

_dot = functools.partial(
    jax.lax.dot_general,
    dimension_numbers=(((1,), (0,)), ((), ())),
    preferred_element_type=jnp.float32)


def _puca_kernel(xz_ref, w2_ref, b2_ref, ws_ref, u4a_ref, u4b_ref,
                 bmid_ref, ftw_ref, bout_ref, o_ref):
    ht = o_ref.shape[2]
    wo = o_ref.shape[3]
    n = ht * wo
    r0 = pl.program_id(1) * ht

    rows_all = xz_ref[0, :, pl.ds(r0, ht + 8), :]
    taps = []
    for kh in range(3):
        for kw in range(3):
            if kh == 1 and kw == 1:
                continue
            taps.append(rows_all[:, 4 + kh:4 + kh + ht, 4 + kw:4 + kw + wo])
    xcol = jnp.concatenate(taps, axis=0)
    xcol = xcol.reshape(8 * rows_all.shape[0], n)

    tcf = _dot(w2_ref[...], xcol) + b2_ref[...]
    skip = _dot(ws_ref[...], xcol)
    tcb = tcf.astype(jnp.bfloat16)

    blk = 2 * wo
    y = jnp.concatenate(
        [jnp.concatenate([tcb[:, (2 * i + a) * blk:(2 * i + a + 1) * blk]
                          for i in range(ht // 4)], axis=1)
         for a in range(2)], axis=0)

    lane = jax.lax.broadcasted_iota(jnp.int32, (256, n // 2), 1)
    laneq = (lane % 4) // 2
    acc = None
    for cc, u_ref in ((0, u4a_ref), (1, u4b_ref)):
        z = _dot(u_ref[...], y)
        for q in range(2):
            zq = z[q * 256:(q + 1) * 256]
            sh = 2 * (q - cc)
            if sh:
                zq = jnp.roll(zq, sh, axis=1)
            zq = jnp.where(laneq == q, zq, 0.0)
            acc = zq if acc is None else acc + zq
    mid = (acc + bmid_ref[...]).astype(jnp.bfloat16)

    pieces = []
    for i in range(ht // 4):
        pieces.append(mid[0:128, i * blk:(i + 1) * blk])
        pieces.append(mid[128:256, i * blk:(i + 1) * blk])
    ymid = jnp.concatenate(pieces, axis=1)

    out = _dot(ftw_ref[...], ymid) + skip + bout_ref[...]
    o_ref[0] = out.reshape(out.shape[0], ht, wo)


def kernel(x, fused_mc_w, fused_mc_b, down0_w, down0_b, up0_w, up0_b,
           fused_tail_w, fused_tail_b):
    B, cimg, H, W = x.shape
    p, mp = 4, 1

    x16 = x.astype(jnp.bfloat16)
    xp = jnp.pad(x16, ((0, 0), (0, 0), (p, p), (p, p)), mode='reflect')
    ones = jnp.ones((B, 1, H + 2 * p, W + 2 * p), jnp.bfloat16)
    xz = jnp.pad(jnp.concatenate([xp, ones], axis=1),
                 ((0, 0), (0, 0), (mp, mp), (mp, mp)))
    ca, hz, wz = cimg + 1, H + 2 * p + 2 * mp, W + 2 * p + 2 * mp
    width = fused_mc_w.shape[0]
    w2c = down0_w.shape[0]

    w2 = down0_w @ fused_mc_w
    b2 = down0_w @ fused_mc_b + down0_b
    ws = fused_tail_w @ fused_mc_w
    bout = fused_tail_w @ fused_mc_b + fused_tail_b
    u6 = up0_w.reshape(width, 2, 2, w2c, 2, 2)
    u4 = u6.transpose(2, 1, 0, 5, 4, 3).reshape(4 * width, 2, 2 * w2c)
    u4a, u4b = u4[:, 0, :], u4[:, 1, :]
    ub = up0_b.reshape(width, 2, 2).transpose(1, 0, 2).reshape(2 * width, 2)
    laneq = (jnp.arange(_TILE_H // 2 * W) % 4) // 2
    bmid = ub[:, laneq]

    return pl.pallas_call(
        _puca_kernel,
        out_shape=jax.ShapeDtypeStruct((B, cimg, H, W), jnp.float32),
        grid=(B, H // _TILE_H),
        in_specs=[
            pl.BlockSpec((1, ca, hz, wz), lambda b, t: (b, 0, 0, 0)),
            pl.BlockSpec(w2.shape, lambda b, t: (0, 0)),
            pl.BlockSpec((w2c, 1), lambda b, t: (0, 0)),
            pl.BlockSpec(ws.shape, lambda b, t: (0, 0)),
            pl.BlockSpec(u4a.shape, lambda b, t: (0, 0)),
            pl.BlockSpec(u4b.shape, lambda b, t: (0, 0)),
            pl.BlockSpec(bmid.shape, lambda b, t: (0, 0)),
            pl.BlockSpec(fused_tail_w.shape, lambda b, t: (0, 0)),
            pl.BlockSpec((cimg, 1), lambda b, t: (0, 0)),
        ],
        out_specs=pl.BlockSpec((1, cimg, _TILE_H, W), lambda b, t: (b, 0, t, 0)),
        compiler_params=pltpu.CompilerParams(
            dimension_semantics=("parallel", "arbitrary")),
    )(xz, w2.astype(jnp.bfloat16), b2.reshape(w2c, 1),
      ws.astype(jnp.bfloat16), u4a.astype(jnp.bfloat16),
      u4b.astype(jnp.bfloat16), bmid,
      fused_tail_w.astype(jnp.bfloat16), bout.reshape(cimg, 1))


# speedup vs baseline: 20.7499x; 1.0707x over previous
import functools

import jax
import jax.numpy as jnp
from jax.experimental import pallas as pl
from jax.experimental.pallas import tpu as pltpu


_TILE_H = 32

_dot = functools.partial(
    jax.lax.dot_general,
    dimension_numbers=(((1,), (0,)), ((), ())),
    preferred_element_type=jnp.float32)


def _puca_kernel(xz_ref, w2_ref, b2_ref, ws_ref, u4a_ref, u4b_ref,
                 bmid_ref, ftw_ref, bout_ref, o_ref):
    ht = o_ref.shape[2]
    wo = o_ref.shape[3]
    n = ht * wo
    r0 = pl.program_id(1) * ht

    rows_all = xz_ref[0, :, pl.ds(r0, ht + 8), :]
    taps = []
    for kh in range(3):
        for kw in range(3):
            if kh == 1 and kw == 1:
                continue
            taps.append(rows_all[:, 4 + kh:4 + kh + ht, 4 + kw:4 + kw + wo])
    xcol = jnp.concatenate(taps, axis=0)
    xcol = xcol.reshape(8 * rows_all.shape[0], n)

    tcf = _dot(w2_ref[...], xcol) + b2_ref[...]
    skip = _dot(ws_ref[...], xcol)
    tcb = tcf.astype(jnp.bfloat16)

    blk = 2 * wo
    y = jnp.concatenate(
        [jnp.concatenate([tcb[:, (2 * i + a) * blk:(2 * i + a + 1) * blk]
                          for i in range(ht // 4)], axis=1)
         for a in range(2)], axis=0)

    lane = jax.lax.broadcasted_iota(jnp.int32, (256, n // 2), 1)
    laneq = (lane % 4) // 2
    acc = None
    for cc, u_ref in ((0, u4a_ref), (1, u4b_ref)):
        z = _dot(u_ref[...], y)
        for q in range(2):
            zq = z[q * 256:(q + 1) * 256]
            sh = 2 * (q - cc)
            if sh:
                zq = jnp.roll(zq, sh, axis=1)
            zq = jnp.where(laneq == q, zq, 0.0)
            acc = zq if acc is None else acc + zq
    mid = (acc + bmid_ref[...]).astype(jnp.bfloat16)

    pieces = []
    for i in range(ht // 4):
        pieces.append(mid[0:128, i * blk:(i + 1) * blk])
        pieces.append(mid[128:256, i * blk:(i + 1) * blk])
    ymid = jnp.concatenate(pieces, axis=1)

    out = _dot(ftw_ref[...], ymid) + skip + bout_ref[...]
    o_ref[0] = out.reshape(out.shape[0], ht, wo)


def kernel(x, fused_mc_w, fused_mc_b, down0_w, down0_b, up0_w, up0_b,
           fused_tail_w, fused_tail_b):
    B, cimg, H, W = x.shape
    p, mp = 4, 1

    x16 = x.astype(jnp.bfloat16)
    xp = jnp.pad(x16, ((0, 0), (0, 0), (p, p), (p, p)), mode='reflect')
    ones = jnp.ones((B, 1, H + 2 * p, W + 2 * p), jnp.bfloat16)
    xz = jnp.pad(jnp.concatenate([xp, ones], axis=1),
                 ((0, 0), (0, 0), (mp, mp), (mp, mp)))
    ca, hz, wz = cimg + 1, H + 2 * p + 2 * mp, W + 2 * p + 2 * mp
    width = fused_mc_w.shape[0]
    w2c = down0_w.shape[0]

    w2 = down0_w @ fused_mc_w
    b2 = down0_w @ fused_mc_b + down0_b
    ws = fused_tail_w @ fused_mc_w
    bout = fused_tail_w @ fused_mc_b + fused_tail_b
    u6 = up0_w.reshape(width, 2, 2, w2c, 2, 2)
    u4 = u6.transpose(2, 1, 0, 5, 4, 3).reshape(4 * width, 2, 2 * w2c)
    u4a, u4b = u4[:, 0, :], u4[:, 1, :]
    ub = up0_b.reshape(width, 2, 2).transpose(1, 0, 2).reshape(2 * width, 2)
    laneq = (jnp.arange(_TILE_H // 2 * W) % 4) // 2
    bmid = ub[:, laneq]

    return pl.pallas_call(
        _puca_kernel,
        out_shape=jax.ShapeDtypeStruct((B, cimg, H, W), jnp.float32),
        grid=(B, H // _TILE_H),
        in_specs=[
            pl.BlockSpec((1, ca, hz, wz), lambda b, t: (b, 0, 0, 0)),
            pl.BlockSpec(w2.shape, lambda b, t: (0, 0)),
            pl.BlockSpec((w2c, 1), lambda b, t: (0, 0)),
            pl.BlockSpec(ws.shape, lambda b, t: (0, 0)),
            pl.BlockSpec(u4a.shape, lambda b, t: (0, 0)),
            pl.BlockSpec(u4b.shape, lambda b, t: (0, 0)),
            pl.BlockSpec(bmid.shape, lambda b, t: (0, 0)),
            pl.BlockSpec(fused_tail_w.shape, lambda b, t: (0, 0)),
            pl.BlockSpec((cimg, 1), lambda b, t: (0, 0)),
        ],
        out_specs=pl.BlockSpec((1, cimg, _TILE_H, W), lambda b, t: (b, 0, t, 0)),
        compiler_params=pltpu.CompilerParams(
            dimension_semantics=("parallel", "arbitrary")),
    )(xz, w2.astype(jnp.bfloat16), b2.reshape(w2c, 1),
      ws.astype(jnp.bfloat16), u4a.astype(jnp.bfloat16),
      u4b.astype(jnp.bfloat16), bmid,
      fused_tail_w.astype(jnp.bfloat16), bout.reshape(cimg, 1))
